# baseline (speedup 1.0000x reference)
import functools

import jax
import jax.numpy as jnp
from jax import lax
from jax.experimental import pallas as pl
from jax.experimental.pallas import tpu as pltpu
from jax.experimental.pallas import tpu_sc as plsc

ROWS = 128
COLS = 32768
LANES = 16
NUM_CORES = 2
NUM_SUBCORES = 16
NUM_WORKERS = NUM_CORES * NUM_SUBCORES
TROW = 8
SC_TROWS = 4
SC_ROWS = SC_TROWS * TROW
SEGS = NUM_WORKERS // SC_TROWS
SEG = COLS // SEGS
CHUNK = 2048
CHUNKS = SEG // CHUNK
STEPS = CHUNK // LANES
NBUF = 4

_INT_MAX = 2147483647


def _argmin_body(x_hbm, val_hbm, idx_hbm, buf, outv_val, outv_idx,
                 sem0, sem1, sem2, sem3):
    sems = (sem0, sem1, sem2, sem3)
    wid = lax.axis_index("s") * NUM_CORES + lax.axis_index("c")
    trow = wid // SEGS
    seg = wid % SEGS
    row0 = trow * TROW
    col0 = seg * SEG
    iota = lax.iota(jnp.int32, LANES)

    def start(c):
        return pltpu.async_copy(
            x_hbm.at[pl.ds(row0, TROW), pl.ds(col0 + c * CHUNK, CHUNK)],
            buf.at[c % NBUF], sems[c % NBUF])

    copies = [None] * NBUF
    for c in range(min(NBUF - 1, CHUNKS)):
        copies[c] = start(c)

    accv = [jnp.full((LANES,), jnp.inf, jnp.float32) for _ in range(TROW)]
    accs = [jnp.zeros((LANES,), jnp.int32) for _ in range(TROW)]

    for c in range(CHUNKS):
        b = c % NBUF
        if c + NBUF - 1 < CHUNKS:
            copies[(c + NBUF - 1) % NBUF] = start(c + NBUF - 1)
        copies[b].wait()

        def p1_body(k, carry, b=b, c=c):
            vs = list(carry[0])
            ss = list(carry[1])
            stamp = jnp.zeros((LANES,), jnp.int32) + (c * STEPS + k)
            for s in range(TROW):
                v = buf[b, s, pl.ds(k * LANES, LANES)]
                m = v < vs[s]
                vs[s] = jnp.where(m, v, vs[s])
                ss[s] = jnp.where(m, stamp, ss[s])
            return (tuple(vs), tuple(ss))

        accv_t, accs_t = plsc.parallel_loop(
            0, STEPS, 1, carry=(tuple(accv), tuple(accs)))(p1_body)
        accv = list(accv_t)
        accs = list(accs_t)

    val_v = jnp.zeros((LANES,), jnp.float32)
    idx_v = jnp.zeros((LANES,), jnp.int32)
    for s in range(TROW):
        rowmin = jnp.min(accv[s])
        colvec = accs[s] * LANES + iota + col0
        cand = jnp.where(accv[s] == rowmin, colvec, jnp.int32(_INT_MAX))
        rowidx = jnp.min(cand)
        val_v = jnp.where(iota == s, rowmin, val_v)
        idx_v = jnp.where(iota == s, rowidx, idx_v)

    outv_val[...] = val_v
    outv_idx[...] = idx_v
    pltpu.sync_copy(outv_val, val_hbm.at[wid])
    pltpu.sync_copy(outv_idx, idx_hbm.at[wid])


def _tc_body(x_ref, out_ref):
    blk = x_ref[...]
    m = jnp.min(blk, axis=1, keepdims=True)
    idx = lax.broadcasted_iota(jnp.int32, blk.shape, 1)
    cand = jnp.where(blk == m, idx, jnp.int32(_INT_MAX))
    mi = jnp.min(cand, axis=1, keepdims=True)
    out_ref[...] = jnp.broadcast_to(mi, (TROW, 128))


def kernel(x):
    mesh = plsc.VectorSubcoreMesh(core_axis_name="c", subcore_axis_name="s")
    sc_k = functools.partial(
        pl.kernel,
        mesh=mesh,
        out_type=(
            jax.ShapeDtypeStruct((NUM_WORKERS, LANES), jnp.float32),
            jax.ShapeDtypeStruct((NUM_WORKERS, LANES), jnp.int32),
        ),
        scratch_types=[
            pltpu.VMEM((NBUF, TROW, CHUNK), jnp.float32),
            pltpu.VMEM((LANES,), jnp.float32),
            pltpu.VMEM((LANES,), jnp.int32),
            pltpu.SemaphoreType.DMA,
            pltpu.SemaphoreType.DMA,
            pltpu.SemaphoreType.DMA,
            pltpu.SemaphoreType.DMA,
        ],
        compiler_params=pltpu.CompilerParams(needs_layout_passes=False),
    )(_argmin_body)
    tc_trows = ROWS // TROW - SC_TROWS
    tc_out = pl.pallas_call(
        _tc_body,
        grid=(tc_trows,),
        in_specs=[pl.BlockSpec((TROW, COLS), lambda i: (i + SC_TROWS, 0))],
        out_specs=pl.BlockSpec((TROW, 128), lambda i: (i, 0)),
        out_shape=jax.ShapeDtypeStruct((tc_trows * TROW, 128), jnp.int32),
    )(x)
    vals, idxs = sc_k(x)

    v = vals[:, :TROW].reshape(SC_TROWS, SEGS, TROW)
    i = idxs[:, :TROW].reshape(SC_TROWS, SEGS, TROW)
    mv, mi = v[:, 0], i[:, 0]
    for s in range(1, SEGS):
        take = v[:, s] < mv
        mv = jnp.where(take, v[:, s], mv)
        mi = jnp.where(take, i[:, s], mi)
    y_sc = mi.reshape(SC_ROWS)
    y_tc = tc_out[:, 0]
    return jnp.concatenate([y_sc, y_tc]).reshape(ROWS, 1)


# speedup vs baseline: 1.1455x; 1.1455x over previous
import functools

import jax
import jax.numpy as jnp
from jax import lax
from jax.experimental import pallas as pl
from jax.experimental.pallas import tpu as pltpu
from jax.experimental.pallas import tpu_sc as plsc

ROWS = 128
COLS = 32768
LANES = 16
NUM_CORES = 2
NUM_SUBCORES = 16
NUM_WORKERS = NUM_CORES * NUM_SUBCORES
TROW = 8
SC_TROWS = 8
SC_ROWS = SC_TROWS * TROW
SEGS = NUM_WORKERS // SC_TROWS
SEG = COLS // SEGS
CHUNK = 2048
CHUNKS = SEG // CHUNK
STEPS = CHUNK // LANES
NBUF = 4

_INT_MAX = 2147483647


def _argmin_body(x_hbm, val_hbm, idx_hbm, buf, outv_val, outv_idx,
                 sem0, sem1, sem2, sem3):
    sems = (sem0, sem1, sem2, sem3)
    wid = lax.axis_index("s") * NUM_CORES + lax.axis_index("c")
    trow = wid // SEGS
    seg = wid % SEGS
    row0 = trow * TROW
    col0 = seg * SEG
    iota = lax.iota(jnp.int32, LANES)

    def start(c):
        return pltpu.async_copy(
            x_hbm.at[pl.ds(row0, TROW), pl.ds(col0 + c * CHUNK, CHUNK)],
            buf.at[c % NBUF], sems[c % NBUF])

    copies = [None] * NBUF
    for c in range(min(NBUF - 1, CHUNKS)):
        copies[c] = start(c)

    accv = [jnp.full((LANES,), jnp.inf, jnp.float32) for _ in range(TROW)]
    accs = [jnp.zeros((LANES,), jnp.int32) for _ in range(TROW)]

    for c in range(CHUNKS):
        b = c % NBUF
        if c + NBUF - 1 < CHUNKS:
            copies[(c + NBUF - 1) % NBUF] = start(c + NBUF - 1)
        copies[b].wait()

        def p1_body(k, carry, b=b, c=c):
            vs = list(carry[0])
            ss = list(carry[1])
            stamp = jnp.zeros((LANES,), jnp.int32) + (c * STEPS + k)
            for s in range(TROW):
                v = buf[b, s, pl.ds(k * LANES, LANES)]
                m = v < vs[s]
                vs[s] = jnp.where(m, v, vs[s])
                ss[s] = jnp.where(m, stamp, ss[s])
            return (tuple(vs), tuple(ss))

        accv_t, accs_t = plsc.parallel_loop(
            0, STEPS, 1, carry=(tuple(accv), tuple(accs)))(p1_body)
        accv = list(accv_t)
        accs = list(accs_t)

    val_v = jnp.zeros((LANES,), jnp.float32)
    idx_v = jnp.zeros((LANES,), jnp.int32)
    for s in range(TROW):
        rowmin = jnp.min(accv[s])
        colvec = accs[s] * LANES + iota + col0
        cand = jnp.where(accv[s] == rowmin, colvec, jnp.int32(_INT_MAX))
        rowidx = jnp.min(cand)
        val_v = jnp.where(iota == s, rowmin, val_v)
        idx_v = jnp.where(iota == s, rowidx, idx_v)

    outv_val[...] = val_v
    outv_idx[...] = idx_v
    pltpu.sync_copy(outv_val, val_hbm.at[wid])
    pltpu.sync_copy(outv_idx, idx_hbm.at[wid])


def _tc_body(x_ref, out_ref):
    blk = x_ref[...]
    m = jnp.min(blk, axis=1, keepdims=True)
    idx = lax.broadcasted_iota(jnp.int32, blk.shape, 1)
    cand = jnp.where(blk == m, idx, jnp.int32(_INT_MAX))
    mi = jnp.min(cand, axis=1, keepdims=True)
    out_ref[...] = jnp.broadcast_to(mi, (TROW, 128))


def kernel(x):
    mesh = plsc.VectorSubcoreMesh(core_axis_name="c", subcore_axis_name="s")
    sc_k = functools.partial(
        pl.kernel,
        mesh=mesh,
        out_type=(
            jax.ShapeDtypeStruct((NUM_WORKERS, LANES), jnp.float32),
            jax.ShapeDtypeStruct((NUM_WORKERS, LANES), jnp.int32),
        ),
        scratch_types=[
            pltpu.VMEM((NBUF, TROW, CHUNK), jnp.float32),
            pltpu.VMEM((LANES,), jnp.float32),
            pltpu.VMEM((LANES,), jnp.int32),
            pltpu.SemaphoreType.DMA,
            pltpu.SemaphoreType.DMA,
            pltpu.SemaphoreType.DMA,
            pltpu.SemaphoreType.DMA,
        ],
        compiler_params=pltpu.CompilerParams(
            needs_layout_passes=False,
            skip_device_barrier=True,
            disable_bounds_checks=True,
            disable_semaphore_checks=True,
        ),
    )(_argmin_body)
    tc_trows = ROWS // TROW - SC_TROWS
    tc_out = pl.pallas_call(
        _tc_body,
        grid=(tc_trows,),
        in_specs=[pl.BlockSpec((TROW, COLS), lambda i: (i + SC_TROWS, 0))],
        out_specs=pl.BlockSpec((TROW, 128), lambda i: (i, 0)),
        out_shape=jax.ShapeDtypeStruct((tc_trows * TROW, 128), jnp.int32),
    )(x)
    vals, idxs = sc_k(x)

    v = vals[:, :TROW].reshape(SC_TROWS, SEGS, TROW)
    i = idxs[:, :TROW].reshape(SC_TROWS, SEGS, TROW)
    mv, mi = v[:, 0], i[:, 0]
    for s in range(1, SEGS):
        take = v[:, s] < mv
        mv = jnp.where(take, v[:, s], mv)
        mi = jnp.where(take, i[:, s], mi)
    y_sc = mi.reshape(SC_ROWS)
    y_tc = tc_out[:, 0]
    return jnp.concatenate([y_sc, y_tc]).reshape(ROWS, 1)


# speedup vs baseline: 1.1965x; 1.0445x over previous
import functools

import jax
import jax.numpy as jnp
from jax import lax
from jax.experimental import pallas as pl
from jax.experimental.pallas import tpu as pltpu
from jax.experimental.pallas import tpu_sc as plsc

ROWS = 128
COLS = 32768
LANES = 16
NUM_CORES = 1
NUM_SUBCORES = 16
NUM_WORKERS = NUM_CORES * NUM_SUBCORES
TROW = 8
SC_TROWS = 8
SC_ROWS = SC_TROWS * TROW
SEGS = NUM_WORKERS // SC_TROWS
SEG = COLS // SEGS
CHUNK = 2048
CHUNKS = SEG // CHUNK
STEPS = CHUNK // LANES
NBUF = 4

_INT_MAX = 2147483647


def _argmin_body(x_hbm, val_hbm, idx_hbm, buf, outv_val, outv_idx,
                 sem0, sem1, sem2, sem3):
    sems = (sem0, sem1, sem2, sem3)
    wid = lax.axis_index("s") * NUM_CORES + lax.axis_index("c")
    trow = wid // SEGS
    seg = wid % SEGS
    row0 = trow * TROW
    col0 = seg * SEG
    iota = lax.iota(jnp.int32, LANES)

    def start(c):
        return pltpu.async_copy(
            x_hbm.at[pl.ds(row0, TROW), pl.ds(col0 + c * CHUNK, CHUNK)],
            buf.at[c % NBUF], sems[c % NBUF])

    copies = [None] * NBUF
    for c in range(min(NBUF - 1, CHUNKS)):
        copies[c] = start(c)

    accv = [jnp.full((LANES,), jnp.inf, jnp.float32) for _ in range(TROW)]
    accs = [jnp.zeros((LANES,), jnp.int32) for _ in range(TROW)]

    for c in range(CHUNKS):
        b = c % NBUF
        if c + NBUF - 1 < CHUNKS:
            copies[(c + NBUF - 1) % NBUF] = start(c + NBUF - 1)
        copies[b].wait()

        def p1_body(k, carry, b=b, c=c):
            vs = list(carry[0])
            ss = list(carry[1])
            stamp = jnp.zeros((LANES,), jnp.int32) + (c * STEPS + k)
            for s in range(TROW):
                v = buf[b, s, pl.ds(k * LANES, LANES)]
                m = v < vs[s]
                vs[s] = jnp.where(m, v, vs[s])
                ss[s] = jnp.where(m, stamp, ss[s])
            return (tuple(vs), tuple(ss))

        accv_t, accs_t = plsc.parallel_loop(
            0, STEPS, 1, carry=(tuple(accv), tuple(accs)))(p1_body)
        accv = list(accv_t)
        accs = list(accs_t)

    val_v = jnp.zeros((LANES,), jnp.float32)
    idx_v = jnp.zeros((LANES,), jnp.int32)
    for s in range(TROW):
        rowmin = jnp.min(accv[s])
        colvec = accs[s] * LANES + iota + col0
        cand = jnp.where(accv[s] == rowmin, colvec, jnp.int32(_INT_MAX))
        rowidx = jnp.min(cand)
        val_v = jnp.where(iota == s, rowmin, val_v)
        idx_v = jnp.where(iota == s, rowidx, idx_v)

    outv_val[...] = val_v
    outv_idx[...] = idx_v
    pltpu.sync_copy(outv_val, val_hbm.at[wid])
    pltpu.sync_copy(outv_idx, idx_hbm.at[wid])


def _tc_body(x_ref, out_ref):
    blk = x_ref[...]
    m = jnp.min(blk, axis=1, keepdims=True)
    idx = lax.broadcasted_iota(jnp.int32, blk.shape, 1)
    cand = jnp.where(blk == m, idx, jnp.int32(_INT_MAX))
    mi = jnp.min(cand, axis=1, keepdims=True)
    out_ref[...] = jnp.broadcast_to(mi, (TROW, 128))


def kernel(x):
    mesh = plsc.VectorSubcoreMesh(core_axis_name="c", subcore_axis_name="s",
                                  num_cores=NUM_CORES)
    sc_k = functools.partial(
        pl.kernel,
        mesh=mesh,
        out_type=(
            jax.ShapeDtypeStruct((NUM_WORKERS, LANES), jnp.float32),
            jax.ShapeDtypeStruct((NUM_WORKERS, LANES), jnp.int32),
        ),
        scratch_types=[
            pltpu.VMEM((NBUF, TROW, CHUNK), jnp.float32),
            pltpu.VMEM((LANES,), jnp.float32),
            pltpu.VMEM((LANES,), jnp.int32),
            pltpu.SemaphoreType.DMA,
            pltpu.SemaphoreType.DMA,
            pltpu.SemaphoreType.DMA,
            pltpu.SemaphoreType.DMA,
        ],
        compiler_params=pltpu.CompilerParams(
            needs_layout_passes=False,
            skip_device_barrier=True,
            disable_bounds_checks=True,
            disable_semaphore_checks=True,
        ),
    )(_argmin_body)
    tc_trows = ROWS // TROW - SC_TROWS
    tc_out = pl.pallas_call(
        _tc_body,
        grid=(tc_trows,),
        in_specs=[pl.BlockSpec((TROW, COLS), lambda i: (i + SC_TROWS, 0))],
        out_specs=pl.BlockSpec((TROW, 128), lambda i: (i, 0)),
        out_shape=jax.ShapeDtypeStruct((tc_trows * TROW, 128), jnp.int32),
    )(x)
    vals, idxs = sc_k(x)

    v = vals[:, :TROW].reshape(SC_TROWS, SEGS, TROW)
    i = idxs[:, :TROW].reshape(SC_TROWS, SEGS, TROW)
    mv, mi = v[:, 0], i[:, 0]
    for s in range(1, SEGS):
        take = v[:, s] < mv
        mv = jnp.where(take, v[:, s], mv)
        mi = jnp.where(take, i[:, s], mi)
    y_sc = mi.reshape(SC_ROWS)
    y_tc = tc_out[:, 0]
    return jnp.concatenate([y_sc, y_tc]).reshape(ROWS, 1)
